# baseline (speedup 1.0000x reference)
import functools

import jax
import jax.numpy as jnp
from jax import lax
from jax.experimental import pallas as pl
from jax.experimental.pallas import tpu as pltpu
from jax.experimental.pallas import tpu_sc as plsc

_BATCH = 16384
_DIM = 64
_NREL = 1000
_NC = 2
_NS = 16
_L = 16
_NW = _NC * _NS
_BPW = _BATCH // _NW
_CH = 128
_NCHUNK = _BPW // _CH
_DU = 8


def _sc_body(headT_hbm, idx_hbm, tailT_hbm, table_hbm, out_hbm,
             table_v, idx_v, out_v, h0, h1, t0, t1, sem0, sem1, tsem):
    wid = lax.axis_index("s") * _NC + lax.axis_index("c")
    base = wid * _BPW

    tcopy = pltpu.async_copy(table_hbm, table_v, tsem)
    pltpu.sync_copy(idx_hbm.at[pl.ds(base, _BPW)], idx_v)

    hbufs, tbufs = (h0, h1), (t0, t1)
    sems = (sem0, sem1)
    lane = lax.iota(jnp.int32, _L)

    def fire(c):
        slot = c % 2
        col = base + c * _CH
        return [
            pltpu.async_copy(headT_hbm.at[:, pl.ds(col, _CH)], hbufs[slot],
                             sems[slot]),
            pltpu.async_copy(tailT_hbm.at[:, pl.ds(col, _CH)], tbufs[slot],
                             sems[slot]),
        ]

    inflight = {0: fire(0)}
    tcopy.wait()
    for c in range(_NCHUNK):
        slot = c % 2
        if c + 1 < _NCHUNK:
            inflight[c + 1] = fire(c + 1)
        for cp in inflight.pop(c):
            cp.wait()
        hv, tv = hbufs[slot], tbufs[slot]

        def group(g, _, c=c, hv=hv, tv=tv):
            blane = g * _L + lane
            rela = idx_v[pl.ds(c * _CH + g * _L, _L)] * _DIM

            def dstep(k, accs, hv=hv, tv=tv, blane=blane, rela=rela):
                new = []
                for q in range(_DU):
                    dvec = (lane + (k * _DU + q)) & (_DIM - 1)
                    h = plsc.load_gather(hv, [dvec, blane])
                    t = plsc.load_gather(tv, [dvec, blane])
                    r = plsc.load_gather(table_v, [rela + dvec])
                    new.append(accs[q] + h * r * t)
                return tuple(new)

            z = jnp.zeros((_L,), jnp.float32)
            accs = lax.fori_loop(0, _DIM // _DU, dstep, (z,) * _DU)
            acc = sum(accs[1:], accs[0])
            out_v[pl.ds(c * _CH + g * _L, _L)] = 1.0 / (1.0 + jnp.exp(-acc))
            return 0

        lax.fori_loop(0, _CH // _L, group, 0)

    pltpu.sync_copy(out_v, out_hbm.at[pl.ds(base, _BPW)])


@jax.jit
def _dist_mult_sc(head_e, rel_idx, tail_e, table):
    mesh = plsc.VectorSubcoreMesh(core_axis_name="c", subcore_axis_name="s")
    run = functools.partial(
        pl.kernel,
        out_type=jax.ShapeDtypeStruct((_BATCH,), jnp.float32),
        mesh=mesh,
        compiler_params=pltpu.CompilerParams(needs_layout_passes=False),
        scratch_types=[
            pltpu.VMEM((_NREL * _DIM,), jnp.float32),
            pltpu.VMEM((_BPW,), jnp.int32),
            pltpu.VMEM((_BPW,), jnp.float32),
            pltpu.VMEM((_DIM, _CH), jnp.float32),
            pltpu.VMEM((_DIM, _CH), jnp.float32),
            pltpu.VMEM((_DIM, _CH), jnp.float32),
            pltpu.VMEM((_DIM, _CH), jnp.float32),
            pltpu.SemaphoreType.DMA,
            pltpu.SemaphoreType.DMA,
            pltpu.SemaphoreType.DMA,
        ],
    )(_sc_body)
    return run(head_e.T, rel_idx, tail_e.T, table.reshape(-1))


def kernel(head_e, rel_idx, tail_e, kernel):
    score = _dist_mult_sc(head_e, rel_idx.astype(jnp.int32), tail_e, kernel)
    return score.reshape(1, _BATCH)


# speedup vs baseline: 1.0056x; 1.0056x over previous
import functools

import jax
import jax.numpy as jnp
from jax import lax
from jax.experimental import pallas as pl
from jax.experimental.pallas import tpu as pltpu
from jax.experimental.pallas import tpu_sc as plsc

_BATCH = 16384
_DIM = 64
_NREL = 1000
_NC = 2
_NS = 16
_L = 16
_NW = _NC * _NS
_BPW = _BATCH // _NW
_CH = 128
_NCHUNK = _BPW // _CH
_DU = 4


def _sc_body(headT_hbm, idx_hbm, tailT_hbm, table_hbm, out_hbm,
             table_v, idx_v, out_v, h0, h1, t0, t1, sem0, sem1, tsem):
    wid = lax.axis_index("s") * _NC + lax.axis_index("c")
    base = wid * _BPW

    tcopy = pltpu.async_copy(table_hbm, table_v, tsem)
    pltpu.sync_copy(idx_hbm.at[pl.ds(base, _BPW)], idx_v)

    hbufs, tbufs = (h0, h1), (t0, t1)
    sems = (sem0, sem1)
    lane = lax.iota(jnp.int32, _L)

    def fire(c):
        slot = c % 2
        col = base + c * _CH
        return [
            pltpu.async_copy(headT_hbm.at[:, pl.ds(col, _CH)], hbufs[slot],
                             sems[slot]),
            pltpu.async_copy(tailT_hbm.at[:, pl.ds(col, _CH)], tbufs[slot],
                             sems[slot]),
        ]

    inflight = {0: fire(0)}
    tcopy.wait()
    for c in range(_NCHUNK):
        slot = c % 2
        if c + 1 < _NCHUNK:
            inflight[c + 1] = fire(c + 1)
        for cp in inflight.pop(c):
            cp.wait()
        hv, tv = hbufs[slot], tbufs[slot]

        def group(g, _, c=c, hv=hv, tv=tv):
            blane = g * _L + lane
            rela = idx_v[pl.ds(c * _CH + g * _L, _L)] * _DIM

            def dstep(k, accs, hv=hv, tv=tv, blane=blane, rela=rela):
                new = []
                for q in range(_DU):
                    dvec = (lane + (k * _DU + q)) & (_DIM - 1)
                    h = plsc.load_gather(hv, [dvec, blane])
                    t = plsc.load_gather(tv, [dvec, blane])
                    r = plsc.load_gather(table_v, [rela + dvec])
                    new.append(accs[q] + h * r * t)
                return tuple(new)

            z = jnp.zeros((_L,), jnp.float32)
            accs = lax.fori_loop(0, _DIM // _DU, dstep, (z,) * _DU)
            acc = sum(accs[1:], accs[0])
            out_v[pl.ds(c * _CH + g * _L, _L)] = 1.0 / (1.0 + jnp.exp(-acc))
            return 0

        lax.fori_loop(0, _CH // _L, group, 0)

    pltpu.sync_copy(out_v, out_hbm.at[pl.ds(base, _BPW)])


@jax.jit
def _dist_mult_sc(head_e, rel_idx, tail_e, table):
    mesh = plsc.VectorSubcoreMesh(core_axis_name="c", subcore_axis_name="s")
    run = functools.partial(
        pl.kernel,
        out_type=jax.ShapeDtypeStruct((_BATCH,), jnp.float32),
        mesh=mesh,
        compiler_params=pltpu.CompilerParams(needs_layout_passes=False),
        scratch_types=[
            pltpu.VMEM((_NREL * _DIM,), jnp.float32),
            pltpu.VMEM((_BPW,), jnp.int32),
            pltpu.VMEM((_BPW,), jnp.float32),
            pltpu.VMEM((_DIM, _CH), jnp.float32),
            pltpu.VMEM((_DIM, _CH), jnp.float32),
            pltpu.VMEM((_DIM, _CH), jnp.float32),
            pltpu.VMEM((_DIM, _CH), jnp.float32),
            pltpu.SemaphoreType.DMA,
            pltpu.SemaphoreType.DMA,
            pltpu.SemaphoreType.DMA,
        ],
    )(_sc_body)
    return run(head_e.T, rel_idx, tail_e.T, table.reshape(-1))


def kernel(head_e, rel_idx, tail_e, kernel):
    score = _dist_mult_sc(head_e, rel_idx.astype(jnp.int32), tail_e, kernel)
    return score.reshape(1, _BATCH)
